# baseline (speedup 1.0000x reference)
import jax
import jax.numpy as jnp
from jax import lax
from jax.experimental import pallas as pl
from jax.experimental.pallas import tpu as pltpu

N = 10000
D = 128
K = 5000
M = 16384


def _ce_step(S, I, m, k, pmat):
    G = 64 // m
    dirm = ((pmat & k) == 0).reshape(G, 2, m, 128)[:, 0]
    S4 = S.reshape(G, 2, m, 128)
    I4 = I.reshape(G, 2, m, 128)
    sA, sB = S4[:, 0], S4[:, 1]
    iA, iB = I4[:, 0], I4[:, 1]
    bef = (sA > sB) | ((sA == sB) & (iA < iB))
    swap = bef ^ dirm
    nsA = jnp.where(swap, sB, sA)
    nsB = jnp.where(swap, sA, sB)
    niA = jnp.where(swap, iB, iA)
    niB = jnp.where(swap, iA, iB)
    S = jnp.concatenate([nsA[:, None], nsB[:, None]], axis=1).reshape(128, 128)
    I = jnp.concatenate([niA[:, None], niB[:, None]], axis=1).reshape(128, 128)
    return S, I


def _topk_body(u_ref, w_ref, b_ref, out_ref):
    wrow = w_ref[...]

    rows = []
    for bb in range(128):
        z = lax.dot_general(wrow, u_ref[bb], (((1,), (1,)), ((), ())),
                            preferred_element_type=jnp.float32)
        rows.append(z)
    Z = jnp.concatenate(rows, axis=0)

    io0 = lax.broadcasted_iota(jnp.int32, (128, 128), 0)
    io1 = lax.broadcasted_iota(jnp.int32, (128, 128), 1)
    pmat_O = io1 * 128 + io0
    pmat_T = io0 * 128 + io1

    ST = jnp.tanh(jnp.abs(Z + b_ref[0, 0]) / 100.0)
    ST = jnp.where(pmat_T < N, ST, -1.0)
    S = jnp.transpose(ST)
    I = pmat_O.astype(jnp.float32)

    for lk in range(1, 8):
        k = 1 << lk
        for lj in range(lk - 1, -1, -1):
            S, I = _ce_step(S, I, 1 << lj, k, pmat_O)
    for lk in range(8, 15):
        k = 1 << lk
        S, I = jnp.transpose(S), jnp.transpose(I)
        for lj in range(lk - 1, 6, -1):
            S, I = _ce_step(S, I, 1 << (lj - 7), k, pmat_T)
        S, I = jnp.transpose(S), jnp.transpose(I)
        for lj in range(6, -1, -1):
            S, I = _ce_step(S, I, 1 << lj, k, pmat_O)

    out_ref[...] = jnp.transpose(I).astype(jnp.int32)


def _topk_idx(Xp3, W, b2):
    out = pl.pallas_call(
        _topk_body,
        out_shape=jax.ShapeDtypeStruct((128, 128), jnp.int32),
    )(Xp3, W, b2)
    return out.reshape(-1)[:K]


def kernel(A, X, W, b):
    Xp3 = jnp.pad(X, ((0, M - N), (0, 0))).reshape(128, 128, 128)
    idx = _topk_idx(Xp3, W, b.reshape(1, 1))
    new_X = X[idx, :]
    A2 = A[idx, :][:, idx]
    return (A2, new_X, idx)


# speedup vs baseline: 1.2611x; 1.2611x over previous
import jax
import jax.numpy as jnp
from jax import lax
from jax.experimental import pallas as pl
from jax.experimental.pallas import tpu as pltpu
from jax.experimental.pallas import tpu_sc as plsc

N = 10000
D = 128
K = 5000
M = 16384
KPAD = 5120
NW = 32


def _ce_step(S, I, m, k, pmat):
    G = 64 // m
    dirm = ((pmat & k) == 0).reshape(G, 2, m, 128)[:, 0]
    S4 = S.reshape(G, 2, m, 128)
    I4 = I.reshape(G, 2, m, 128)
    sA, sB = S4[:, 0], S4[:, 1]
    iA, iB = I4[:, 0], I4[:, 1]
    bef = (sA > sB) | ((sA == sB) & (iA < iB))
    swap = bef ^ dirm
    nsA = jnp.where(swap, sB, sA)
    nsB = jnp.where(swap, sA, sB)
    niA = jnp.where(swap, iB, iA)
    niB = jnp.where(swap, iA, iB)
    S = jnp.concatenate([nsA[:, None], nsB[:, None]], axis=1).reshape(128, 128)
    I = jnp.concatenate([niA[:, None], niB[:, None]], axis=1).reshape(128, 128)
    return S, I


def _topk_body(u_ref, w_ref, b_ref, out_ref):
    wrow = w_ref[...]

    rows = []
    for bb in range(128):
        z = lax.dot_general(wrow, u_ref[bb], (((1,), (1,)), ((), ())),
                            preferred_element_type=jnp.float32)
        rows.append(z)
    Z = jnp.concatenate(rows, axis=0)

    io0 = lax.broadcasted_iota(jnp.int32, (128, 128), 0)
    io1 = lax.broadcasted_iota(jnp.int32, (128, 128), 1)
    pmat_O = io1 * 128 + io0
    pmat_T = io0 * 128 + io1

    ST = jnp.tanh(jnp.abs(Z + b_ref[0, 0]) / 100.0)
    ST = jnp.where(pmat_T < N, ST, -1.0)
    S = jnp.transpose(ST)
    I = pmat_O.astype(jnp.float32)

    for lk in range(1, 8):
        k = 1 << lk
        for lj in range(lk - 1, -1, -1):
            S, I = _ce_step(S, I, 1 << lj, k, pmat_O)
    for lk in range(8, 15):
        k = 1 << lk
        S, I = jnp.transpose(S), jnp.transpose(I)
        for lj in range(lk - 1, 6, -1):
            S, I = _ce_step(S, I, 1 << (lj - 7), k, pmat_T)
        S, I = jnp.transpose(S), jnp.transpose(I)
        for lj in range(6, -1, -1):
            S, I = _ce_step(S, I, 1 << lj, k, pmat_O)

    out_ref[...] = jnp.transpose(I).astype(jnp.int32)


def _topk_idx(Xp3, W, b2):
    out = pl.pallas_call(
        _topk_body,
        out_shape=jax.ShapeDtypeStruct((128, 128), jnp.int32),
    )(Xp3, W, b2)
    return out.reshape(-1)[:K]


def _gather_body(idxp_hbm, a_hbm, x_hbm, a2_hbm, nx_hbm,
                 idx_v, row_v, xrow_v, sem_a, sem_x):
    c = lax.axis_index("c")
    s = lax.axis_index("s")
    w = s * 2 + c
    pltpu.sync_copy(idxp_hbm, idx_v)

    nrows = (K - w + NW - 1) // NW

    def row_body(i, _):
        r = w + i * NW
        rowid = idx_v[pl.ds(r, 16)][0]
        src = a_hbm.at[pl.ds(rowid * N, N)].at[idx_v]
        pltpu.async_copy(src, row_v, sem_a).wait()
        pltpu.sync_copy(row_v.at[pl.ds(0, K)], a2_hbm.at[pl.ds(r * K, K)])
        return 0

    lax.fori_loop(0, nrows, row_body, 0)

    nch = (K // 8 - w + NW - 1) // NW

    def x_body(t, _):
        j = w + t * NW
        pltpu.async_copy(x_hbm.at[idx_v.at[pl.ds(j * 8, 8)]], xrow_v, sem_x).wait()
        pltpu.sync_copy(xrow_v, nx_hbm.at[pl.ds(j * 8, 8)])
        return 0

    lax.fori_loop(0, nch, x_body, 0)


def _sc_gather(idxp, A_flat, X):
    mesh = plsc.VectorSubcoreMesh(core_axis_name="c", subcore_axis_name="s")
    f = pl.kernel(
        _gather_body,
        out_type=(
            jax.ShapeDtypeStruct((K * K,), jnp.float32),
            jax.ShapeDtypeStruct((K, D), jnp.float32),
        ),
        mesh=mesh,
        scratch_types=[
            pltpu.VMEM((KPAD,), jnp.int32),
            pltpu.VMEM((KPAD,), jnp.float32),
            pltpu.VMEM((8, D), jnp.float32),
            pltpu.SemaphoreType.DMA,
            pltpu.SemaphoreType.DMA,
        ],
    )
    return f(idxp, A_flat, X)


def kernel(A, X, W, b):
    Xp3 = jnp.pad(X, ((0, M - N), (0, 0))).reshape(128, 128, 128)
    idx = _topk_idx(Xp3, W, b.reshape(1, 1))
    idxp = jnp.concatenate([idx, jnp.broadcast_to(idx[:1], (KPAD - K,))])
    a2_flat, new_X = _sc_gather(idxp, A.reshape(N * N), X)
    return (a2_flat.reshape(K, K), new_X, idx)


# speedup vs baseline: 1.3229x; 1.0490x over previous
import jax
import jax.numpy as jnp
from jax import lax
from jax.experimental import pallas as pl
from jax.experimental.pallas import tpu as pltpu
from jax.experimental.pallas import tpu_sc as plsc

N = 10000
D = 128
K = 5000
M = 16384
KPAD = 5120
NW = 32


def _ce_step(S, I, m, k, pmat):
    G = 64 // m
    dirm = ((pmat & k) == 0).reshape(G, 2, m, 128)[:, 0]
    S4 = S.reshape(G, 2, m, 128)
    I4 = I.reshape(G, 2, m, 128)
    sA, sB = S4[:, 0], S4[:, 1]
    iA, iB = I4[:, 0], I4[:, 1]
    bef = (sA > sB) | ((sA == sB) & (iA < iB))
    swap = bef ^ dirm
    nsA = jnp.where(swap, sB, sA)
    nsB = jnp.where(swap, sA, sB)
    niA = jnp.where(swap, iB, iA)
    niB = jnp.where(swap, iA, iB)
    S = jnp.concatenate([nsA[:, None], nsB[:, None]], axis=1).reshape(128, 128)
    I = jnp.concatenate([niA[:, None], niB[:, None]], axis=1).reshape(128, 128)
    return S, I


def _topk_body(u_ref, w_ref, b_ref, out_ref):
    wrow = w_ref[...]

    rows = []
    for bb in range(128):
        z = lax.dot_general(wrow, u_ref[bb], (((1,), (1,)), ((), ())),
                            preferred_element_type=jnp.float32)
        rows.append(z)
    Z = jnp.concatenate(rows, axis=0)

    io0 = lax.broadcasted_iota(jnp.int32, (128, 128), 0)
    io1 = lax.broadcasted_iota(jnp.int32, (128, 128), 1)
    pmat_O = io1 * 128 + io0
    pmat_T = io0 * 128 + io1

    ST = jnp.tanh(jnp.abs(Z + b_ref[0, 0]) / 100.0)
    ST = jnp.where(pmat_T < N, ST, -1.0)
    S = jnp.transpose(ST)
    I = pmat_O.astype(jnp.float32)

    for lk in range(1, 8):
        k = 1 << lk
        for lj in range(lk - 1, -1, -1):
            S, I = _ce_step(S, I, 1 << lj, k, pmat_O)
    for lk in range(8, 15):
        k = 1 << lk
        S, I = jnp.transpose(S), jnp.transpose(I)
        for lj in range(lk - 1, 6, -1):
            S, I = _ce_step(S, I, 1 << (lj - 7), k, pmat_T)
        S, I = jnp.transpose(S), jnp.transpose(I)
        for lj in range(6, -1, -1):
            S, I = _ce_step(S, I, 1 << lj, k, pmat_O)

    out_ref[...] = jnp.transpose(I).astype(jnp.int32)


def _topk_idx(Xp3, W, b2):
    out = pl.pallas_call(
        _topk_body,
        out_shape=jax.ShapeDtypeStruct((128, 128), jnp.int32),
    )(Xp3, W, b2)
    return out.reshape(-1)[:K]


NB = 4
ROW_IT = 160
XNB = 2
XCH = 8
X_IT = 20


def _gather_body(idxp_hbm, a_hbm, x_hbm, a2_hbm, nx_hbm,
                 idx_v, rb0, rb1, rb2, rb3, xb0, xb1,
                 sg0, sg1, sg2, sg3, ss0, ss1, ss2, ss3,
                 xg0, xg1, xs0, xs1):
    c = lax.axis_index("c")
    s = lax.axis_index("s")
    w = s * 2 + c
    pltpu.sync_copy(idxp_hbm, idx_v)
    rowbuf = (rb0, rb1, rb2, rb3)
    sem_g = (sg0, sg1, sg2, sg3)
    sem_s = (ss0, ss1, ss2, ss3)

    def _row_of(i):
        return jnp.minimum(w + i * NW, K - 1)

    def _fire_gather(i, b):
        r = _row_of(i)
        rowid = idx_v[pl.ds(r, 16)][0]
        return pltpu.async_copy(
            a_hbm.at[pl.ds(rowid * N, N)].at[idx_v], rowbuf[b], sem_g[b])

    def outer(t, _):
        handles = []
        for b in range(NB):
            @pl.when(t > 0)
            def _drain(b=b):
                pltpu.make_async_copy(
                    rowbuf[b].at[pl.ds(0, K)], a2_hbm.at[pl.ds(0, K)],
                    sem_s[b]).wait()
            handles.append(_fire_gather(t * NB + b, b))
        for b in range(NB):
            handles[b].wait()
            r = _row_of(t * NB + b)
            pltpu.async_copy(rowbuf[b].at[pl.ds(0, K)],
                             a2_hbm.at[pl.ds(r * K, K)], sem_s[b])
        return 0

    lax.fori_loop(0, ROW_IT // NB, outer, 0)
    for b in range(NB):
        pltpu.make_async_copy(rowbuf[b].at[pl.ds(0, K)],
                              a2_hbm.at[pl.ds(0, K)], sem_s[b]).wait()

    xbuf = (xb0, xb1)
    xsem_g = (xg0, xg1)
    xsem_s = (xs0, xs1)

    def x_outer(t, _):
        handles = []
        for b in range(XNB):
            j = jnp.minimum(w + (t * XNB + b) * NW, K // XCH - 1)
            @pl.when(t > 0)
            def _drain(b=b):
                pltpu.make_async_copy(xbuf[b], nx_hbm.at[pl.ds(0, XCH)],
                                      xsem_s[b]).wait()
            handles.append(pltpu.async_copy(
                x_hbm.at[idx_v.at[pl.ds(j * XCH, XCH)]], xbuf[b], xsem_g[b]))
        for b in range(XNB):
            handles[b].wait()
            j = jnp.minimum(w + (t * XNB + b) * NW, K // XCH - 1)
            pltpu.async_copy(xbuf[b], nx_hbm.at[pl.ds(j * XCH, XCH)], xsem_s[b])
        return 0

    lax.fori_loop(0, X_IT // XNB, x_outer, 0)
    for b in range(XNB):
        pltpu.make_async_copy(xbuf[b], nx_hbm.at[pl.ds(0, XCH)],
                              xsem_s[b]).wait()


def _sc_gather(idxp, A_flat, X):
    mesh = plsc.VectorSubcoreMesh(core_axis_name="c", subcore_axis_name="s")
    f = pl.kernel(
        _gather_body,
        out_type=(
            jax.ShapeDtypeStruct((K * K,), jnp.float32),
            jax.ShapeDtypeStruct((K, D), jnp.float32),
        ),
        mesh=mesh,
        scratch_types=(
            [pltpu.VMEM((KPAD,), jnp.int32)]
            + [pltpu.VMEM((KPAD,), jnp.float32) for _ in range(NB)]
            + [pltpu.VMEM((XCH, D), jnp.float32) for _ in range(XNB)]
            + [pltpu.SemaphoreType.DMA for _ in range(2 * NB + 2 * XNB)]
        ),
    )
    return f(idxp, A_flat, X)


def kernel(A, X, W, b):
    Xp3 = jnp.pad(X, ((0, M - N), (0, 0))).reshape(128, 128, 128)
    idx = _topk_idx(Xp3, W, b.reshape(1, 1))
    idxp = jnp.concatenate([idx, jnp.broadcast_to(idx[:1], (KPAD - K,))])
    a2_flat, new_X = _sc_gather(idxp, A.reshape(N * N), X)
    return (a2_flat.reshape(K, K), new_X, idx)


# speedup vs baseline: 2.7226x; 2.0581x over previous
import jax
import jax.numpy as jnp
from jax import lax
from jax.experimental import pallas as pl
from jax.experimental.pallas import tpu as pltpu
from jax.experimental.pallas import tpu_sc as plsc

N = 10000
D = 128
K = 5000
M = 16384
KPAD = 5120
NW = 32


def _ce_step(S, I, m, k, pmat):
    G = 64 // m
    dirm = ((pmat & k) == 0).reshape(G, 2, m, 128)[:, 0]
    S4 = S.reshape(G, 2, m, 128)
    I4 = I.reshape(G, 2, m, 128)
    sA, sB = S4[:, 0], S4[:, 1]
    iA, iB = I4[:, 0], I4[:, 1]
    bef = (sA > sB) | ((sA == sB) & (iA < iB))
    swap = bef ^ dirm
    nsA = jnp.where(swap, sB, sA)
    nsB = jnp.where(swap, sA, sB)
    niA = jnp.where(swap, iB, iA)
    niB = jnp.where(swap, iA, iB)
    S = jnp.concatenate([nsA[:, None], nsB[:, None]], axis=1).reshape(128, 128)
    I = jnp.concatenate([niA[:, None], niB[:, None]], axis=1).reshape(128, 128)
    return S, I


def _topk_body(u_ref, w_ref, b_ref, out_ref):
    wrow = w_ref[...]

    rows = []
    for bb in range(128):
        z = lax.dot_general(wrow, u_ref[bb], (((1,), (1,)), ((), ())),
                            preferred_element_type=jnp.float32)
        rows.append(z)
    Z = jnp.concatenate(rows, axis=0)

    io0 = lax.broadcasted_iota(jnp.int32, (128, 128), 0)
    io1 = lax.broadcasted_iota(jnp.int32, (128, 128), 1)
    pmat_O = io1 * 128 + io0
    pmat_T = io0 * 128 + io1

    ST = jnp.tanh(jnp.abs(Z + b_ref[0, 0]) / 100.0)
    ST = jnp.where(pmat_T < N, ST, -1.0)
    S = jnp.transpose(ST)
    I = pmat_O.astype(jnp.float32)

    for lk in range(1, 8):
        k = 1 << lk
        for lj in range(lk - 1, -1, -1):
            S, I = _ce_step(S, I, 1 << lj, k, pmat_O)
    for lk in range(8, 15):
        k = 1 << lk
        S, I = jnp.transpose(S), jnp.transpose(I)
        for lj in range(lk - 1, 6, -1):
            S, I = _ce_step(S, I, 1 << (lj - 7), k, pmat_T)
        S, I = jnp.transpose(S), jnp.transpose(I)
        for lj in range(6, -1, -1):
            S, I = _ce_step(S, I, 1 << lj, k, pmat_O)

    out_ref[...] = jnp.transpose(I).astype(jnp.int32)


def _topk_idx(Xp3, W, b2):
    out = pl.pallas_call(
        _topk_body,
        out_shape=jax.ShapeDtypeStruct((128, 128), jnp.int32),
    )(Xp3, W, b2)
    return out.reshape(-1)[:K]


NB = 4
ROW_IT = 160
XNB = 2
XCH = 8
X_IT = 20


def _gather_body(idxp_hbm, a_hbm, x_hbm, a2_hbm, nx_hbm,
                 idx_v, rf0, rf1, rf2, rf3, ob0, ob1, ob2, ob3, xb0, xb1,
                 sg0, sg1, sg2, sg3, ss0, ss1, ss2, ss3,
                 xg0, xg1, xs0, xs1):
    c = lax.axis_index("c")
    s = lax.axis_index("s")
    w = s * 2 + c
    pltpu.sync_copy(idxp_hbm, idx_v)
    rowbuf = (rf0, rf1, rf2, rf3)
    outbuf = (ob0, ob1, ob2, ob3)
    sem_g = (sg0, sg1, sg2, sg3)
    sem_s = (ss0, ss1, ss2, ss3)

    def _row_of(i):
        return jnp.minimum(w + i * NW, K - 1)

    def _fire_row_fetch(i, b):
        r = _row_of(i)
        rowid = idx_v[pl.ds(r, 16)][0]
        pltpu.async_copy(a_hbm.at[pl.ds(rowid * N, N)],
                         rowbuf[b].at[pl.ds(0, N)], sem_g[b])

    def _compact(b):
        def inner(q0, _):
            for qq in range(16):
                q = q0 * 16 + qq
                cols = idx_v[pl.ds(q * 16, 16)]
                vals = plsc.load_gather(rowbuf[b], [cols])
                outbuf[b][pl.ds(q * 16, 16)] = vals
            return 0
        lax.fori_loop(0, KPAD // 256, inner, 0)

    for b in range(NB):
        _fire_row_fetch(b, b)

    def outer(t, _):
        for b in range(NB):
            i = t * NB + b
            @pl.when(t > 0)
            def _drain_scatter(b=b):
                pltpu.make_async_copy(outbuf[b].at[pl.ds(0, K)],
                                      a2_hbm.at[pl.ds(0, K)], sem_s[b]).wait()
            pltpu.make_async_copy(a_hbm.at[pl.ds(0, N)],
                                  rowbuf[b].at[pl.ds(0, N)], sem_g[b]).wait()
            _compact(b)
            r = _row_of(i)
            pltpu.async_copy(outbuf[b].at[pl.ds(0, K)],
                             a2_hbm.at[pl.ds(r * K, K)], sem_s[b])
            _fire_row_fetch(jnp.minimum(i + NB, ROW_IT - 1), b)
        return 0

    lax.fori_loop(0, ROW_IT // NB, outer, 0)
    for b in range(NB):
        pltpu.make_async_copy(a_hbm.at[pl.ds(0, N)],
                              rowbuf[b].at[pl.ds(0, N)], sem_g[b]).wait()
        pltpu.make_async_copy(outbuf[b].at[pl.ds(0, K)],
                              a2_hbm.at[pl.ds(0, K)], sem_s[b]).wait()

    xbuf = (xb0, xb1)
    xsem_g = (xg0, xg1)
    xsem_s = (xs0, xs1)

    def x_outer(t, _):
        handles = []
        for b in range(XNB):
            j = jnp.minimum(w + (t * XNB + b) * NW, K // XCH - 1)
            @pl.when(t > 0)
            def _drain(b=b):
                pltpu.make_async_copy(xbuf[b], nx_hbm.at[pl.ds(0, XCH)],
                                      xsem_s[b]).wait()
            handles.append(pltpu.async_copy(
                x_hbm.at[idx_v.at[pl.ds(j * XCH, XCH)]], xbuf[b], xsem_g[b]))
        for b in range(XNB):
            handles[b].wait()
            j = jnp.minimum(w + (t * XNB + b) * NW, K // XCH - 1)
            pltpu.async_copy(xbuf[b], nx_hbm.at[pl.ds(j * XCH, XCH)], xsem_s[b])
        return 0

    lax.fori_loop(0, X_IT // XNB, x_outer, 0)
    for b in range(XNB):
        pltpu.make_async_copy(xbuf[b], nx_hbm.at[pl.ds(0, XCH)],
                              xsem_s[b]).wait()


def _sc_gather(idxp, A_flat, X):
    mesh = plsc.VectorSubcoreMesh(core_axis_name="c", subcore_axis_name="s")
    f = pl.kernel(
        _gather_body,
        out_type=(
            jax.ShapeDtypeStruct((K * K,), jnp.float32),
            jax.ShapeDtypeStruct((K, D), jnp.float32),
        ),
        mesh=mesh,
        compiler_params=pltpu.CompilerParams(needs_layout_passes=False),
        scratch_types=(
            [pltpu.VMEM((KPAD,), jnp.int32)]
            + [pltpu.VMEM((N + 240,), jnp.float32) for _ in range(NB)]
            + [pltpu.VMEM((KPAD,), jnp.float32) for _ in range(NB)]
            + [pltpu.VMEM((XCH, D), jnp.float32) for _ in range(XNB)]
            + [pltpu.SemaphoreType.DMA for _ in range(2 * NB + 2 * XNB)]
        ),
    )
    return f(idxp, A_flat, X)


def kernel(A, X, W, b):
    Xp3 = jnp.pad(X, ((0, M - N), (0, 0))).reshape(128, 128, 128)
    idx = _topk_idx(Xp3, W, b.reshape(1, 1))
    idxp = jnp.concatenate([idx, jnp.broadcast_to(idx[:1], (KPAD - K,))])
    a2_flat, new_X = _sc_gather(idxp, A.reshape(N * N), X)
    return (a2_flat.reshape(K, K), new_X, idx)


# speedup vs baseline: 8.1001x; 2.9751x over previous
import jax
import jax.numpy as jnp
from jax import lax
from jax.experimental import pallas as pl
from jax.experimental.pallas import tpu as pltpu
from jax.experimental.pallas import tpu_sc as plsc

N = 10000
D = 128
K = 5000
M = 16384
KPAD = 5120
NW = 32


def _ce_step(S, I, m, k, pmat):
    G = 64 // m
    dirm = ((pmat & k) == 0).reshape(G, 2, m, 128)[:, 0]
    S4 = S.reshape(G, 2, m, 128)
    I4 = I.reshape(G, 2, m, 128)
    sA, sB = S4[:, 0], S4[:, 1]
    iA, iB = I4[:, 0], I4[:, 1]
    bef = (sA > sB) | ((sA == sB) & (iA < iB))
    swap = bef ^ dirm
    nsA = jnp.where(swap, sB, sA)
    nsB = jnp.where(swap, sA, sB)
    niA = jnp.where(swap, iB, iA)
    niB = jnp.where(swap, iA, iB)
    S = jnp.concatenate([nsA[:, None], nsB[:, None]], axis=1).reshape(128, 128)
    I = jnp.concatenate([niA[:, None], niB[:, None]], axis=1).reshape(128, 128)
    return S, I


def _topk_body(x_ref, w_ref, b_ref, out_ref):
    wrow = w_ref[...]

    rows = []
    for bb in range(128):
        if bb < 78:
            xb = x_ref[pl.ds(bb * 128, 128), :]
            z = lax.dot_general(wrow, xb, (((1,), (1,)), ((), ())),
                                preferred_element_type=jnp.float32)
        elif bb == 78:
            xb = x_ref[pl.ds(78 * 128, N - 78 * 128), :]
            z16 = lax.dot_general(wrow, xb, (((1,), (1,)), ((), ())),
                                  preferred_element_type=jnp.float32)
            z = jnp.pad(z16, ((0, 0), (0, 112)))
        else:
            z = jnp.zeros((1, 128), jnp.float32)
        rows.append(z)
    Z = jnp.concatenate(rows, axis=0)

    io0 = lax.broadcasted_iota(jnp.int32, (128, 128), 0)
    io1 = lax.broadcasted_iota(jnp.int32, (128, 128), 1)
    pmat_O = io1 * 128 + io0
    pmat_T = io0 * 128 + io1

    ST = jnp.tanh(jnp.abs(Z + b_ref[0, 0]) / 100.0)
    ST = jnp.where(pmat_T < N, ST, -1.0)
    S = jnp.transpose(ST)
    I = pmat_O.astype(jnp.float32)

    for lk in range(1, 8):
        k = 1 << lk
        for lj in range(lk - 1, -1, -1):
            S, I = _ce_step(S, I, 1 << lj, k, pmat_O)
    for lk in range(8, 15):
        k = 1 << lk
        S, I = jnp.transpose(S), jnp.transpose(I)
        for lj in range(lk - 1, 6, -1):
            S, I = _ce_step(S, I, 1 << (lj - 7), k, pmat_T)
        S, I = jnp.transpose(S), jnp.transpose(I)
        for lj in range(6, -1, -1):
            S, I = _ce_step(S, I, 1 << lj, k, pmat_O)

    out_ref[...] = jnp.transpose(I).astype(jnp.int32)


def _topk_idx(X, W, b2):
    out = pl.pallas_call(
        _topk_body,
        out_shape=jax.ShapeDtypeStruct((128, 128), jnp.int32),
    )(X, W, b2)
    return out.reshape(-1)[:K]


RNB = 4
ONB = 2
NG = 625
G_IT = 20
KC = 5008
XNB = 2
XCH = 8
X_IT = 20


def _gather_body(idxp_hbm, a_hbm, x_hbm, a2_hbm, nx_hbm,
                 idx_v, rf0, rf1, rf2, rf3, gb0, gb1, xb0, xb1,
                 sg0, sg1, sg2, sg3, ss0, ss1,
                 xg0, xg1, xs0, xs1):
    c = lax.axis_index("c")
    s = lax.axis_index("s")
    w = s * 2 + c
    pltpu.sync_copy(idxp_hbm, idx_v)
    rowbuf = (rf0, rf1, rf2, rf3)
    grpbuf = (gb0, gb1)
    sem_g = (sg0, sg1, sg2, sg3)
    sem_s = (ss0, ss1)

    def _grp_of(t):
        return jnp.minimum(w + t * NW, NG - 1)

    def _fire_row_fetch(t, sub, rb):
        r = 8 * _grp_of(t) + sub
        rowid = idx_v[pl.ds(r, 16)][0]
        pltpu.async_copy(a_hbm.at[pl.ds(rowid, 1), :],
                         rowbuf[rb].at[pl.ds(0, 1), :], sem_g[rb])

    def _compact(rb, ob, sub):
        zero16 = jnp.zeros((16,), jnp.int32)

        @plsc.parallel_loop(0, KC // 16, unroll=8)
        def _inner(q):
            cols = idx_v[pl.ds(q * 16, 16)]
            vals = plsc.load_gather(rowbuf[rb], [zero16, cols])
            grpbuf[ob][pl.ds(sub * K + q * 16, 16)] = vals

    for rb in range(RNB):
        _fire_row_fetch(0, rb, rb)

    def outer(T, _):
        for tt in range(ONB):
            t = T * ONB + tt
            ob = tt
            @pl.when(T > 0)
            def _drain_scatter(ob=ob):
                pltpu.make_async_copy(grpbuf[ob].at[pl.ds(0, 8 * K)],
                                      a2_hbm.at[pl.ds(0, 8 * K)], sem_s[ob]).wait()
            for sub in range(8):
                rb = sub % RNB
                pltpu.make_async_copy(a_hbm.at[pl.ds(0, 1), :],
                                      rowbuf[rb].at[pl.ds(0, 1), :],
                                      sem_g[rb]).wait()
                _compact(rb, ob, sub)
                if sub < RNB:
                    _fire_row_fetch(t, sub + RNB, rb)
                else:
                    _fire_row_fetch(t + 1, sub - RNB, rb)
            g = _grp_of(t)
            pltpu.async_copy(grpbuf[ob].at[pl.ds(0, 8 * K)],
                             a2_hbm.at[pl.ds(8 * g * K, 8 * K)], sem_s[ob])
        return 0

    lax.fori_loop(0, G_IT // ONB, outer, 0)
    for rb in range(RNB):
        pltpu.make_async_copy(a_hbm.at[pl.ds(0, 1), :],
                              rowbuf[rb].at[pl.ds(0, 1), :], sem_g[rb]).wait()
    for ob in range(ONB):
        pltpu.make_async_copy(grpbuf[ob].at[pl.ds(0, 8 * K)],
                              a2_hbm.at[pl.ds(0, 8 * K)], sem_s[ob]).wait()

    xbuf = (xb0, xb1)
    xsem_g = (xg0, xg1)
    xsem_s = (xs0, xs1)

    def x_outer(t, _):
        handles = []
        for b in range(XNB):
            j = jnp.minimum(w + (t * XNB + b) * NW, K // XCH - 1)
            @pl.when(t > 0)
            def _drain(b=b):
                pltpu.make_async_copy(xbuf[b], nx_hbm.at[pl.ds(0, XCH)],
                                      xsem_s[b]).wait()
            handles.append(pltpu.async_copy(
                x_hbm.at[idx_v.at[pl.ds(j * XCH, XCH)]], xbuf[b], xsem_g[b]))
        for b in range(XNB):
            handles[b].wait()
            j = jnp.minimum(w + (t * XNB + b) * NW, K // XCH - 1)
            pltpu.async_copy(xbuf[b], nx_hbm.at[pl.ds(j * XCH, XCH)], xsem_s[b])
        return 0

    lax.fori_loop(0, X_IT // XNB, x_outer, 0)
    for b in range(XNB):
        pltpu.make_async_copy(xbuf[b], nx_hbm.at[pl.ds(0, XCH)],
                              xsem_s[b]).wait()


def _sc_gather(idxp, A, X):
    mesh = plsc.VectorSubcoreMesh(core_axis_name="c", subcore_axis_name="s")
    f = pl.kernel(
        _gather_body,
        out_type=(
            jax.ShapeDtypeStruct((K * K,), jnp.float32),
            jax.ShapeDtypeStruct((K, D), jnp.float32),
        ),
        mesh=mesh,
        compiler_params=pltpu.CompilerParams(needs_layout_passes=False),
        scratch_types=(
            [pltpu.VMEM((KPAD,), jnp.int32)]
            + [pltpu.VMEM((1, N), jnp.float32) for _ in range(RNB)]
            + [pltpu.VMEM((8 * K + 64,), jnp.float32) for _ in range(ONB)]
            + [pltpu.VMEM((XCH, D), jnp.float32) for _ in range(XNB)]
            + [pltpu.SemaphoreType.DMA for _ in range(RNB + ONB + 2 * XNB)]
        ),
    )
    return f(idxp, A, X)


def kernel(A, X, W, b):
    idx = _topk_idx(X, W, b.reshape(1, 1))
    idxp = jnp.concatenate([idx, jnp.broadcast_to(idx[:1], (KPAD - K,))])
    a2_flat, new_X = _sc_gather(idxp, A, X)
    return (a2_flat.reshape(K, K), new_X, idx)
